# baseline (speedup 1.0000x reference)
import functools

import jax
import jax.numpy as jnp
from jax import lax
from jax.experimental import pallas as pl
from jax.experimental.pallas import tpu as pltpu
from jax.experimental.pallas import tpu_sc as plsc

_NC = 2
_NS = 16
_NW = _NC * _NS
_L = 16
_CHUNK = 16384
_UNROLL = 8


def _splat(tab_ref, idx):
    return plsc.load_gather(tab_ref, [jnp.full((_L,), idx, jnp.int32)])


def _make_body(npts):
    def body(xs_hbm, xctl_hbm, yctl_hbm, out_hbm,
             xtab, ytab, c0tab, c1tab, c2tab, c3tab,
             xin0, xin1, yout0, yout1,
             sem_in0, sem_in1, sem_out0, sem_out1):
        n = out_hbm.shape[0]
        per_worker = n // _NW
        n_chunks = per_worker // _CHUNK
        wid = lax.axis_index("s") * _NC + lax.axis_index("c")
        base = wid * per_worker

        pltpu.sync_copy(xctl_hbm, xtab)
        pltpu.sync_copy(yctl_hbm, ytab)

        k = lax.iota(jnp.int32, _L)
        kc = jnp.minimum(k, npts - 1)
        kp1 = jnp.minimum(k + 1, npts - 1)
        xv = plsc.load_gather(xtab, [kc])
        yv = plsc.load_gather(ytab, [kc])
        xn = plsc.load_gather(xtab, [kp1])
        yn = plsc.load_gather(ytab, [kp1])
        s = (yn - yv) / (xn - xv)
        c0tab[...] = s
        s_last = _splat(c0tab, npts - 2)
        s = jnp.where(k >= npts - 1, s_last, s)
        c0tab[...] = s
        s_prev = plsc.load_gather(c0tab, [jnp.maximum(k - 1, 0)])
        m = (s + s_prev) * 0.5
        c0tab[...] = m
        m_next = plsc.load_gather(c0tab, [kp1])

        x0 = _splat(xtab, 0)
        dx = _splat(xtab, 1) - x0
        inv_dx = 1.0 / dx

        a = m * dx
        b = m_next * dx
        dy = yn - yv
        c0v = yv
        c1v = a
        c0tab[...] = c0v
        c1tab[...] = c1v
        pad_ok = k < npts - 1
        c2tab[...] = jnp.where(pad_ok, 3.0 * dy - 2.0 * a - b, 0.0)
        c3tab[...] = jnp.where(pad_ok, a + b - 2.0 * dy, 0.0)
        c2v = c2tab[...]
        c3v = c3tab[...]
        bmax = jnp.full((_L,), float(npts - 2), jnp.float32)

        def compute(src, dst):
            @plsc.parallel_loop(0, _CHUNK, step=_L, unroll=_UNROLL)
            def _vecs(i):
                x = src[pl.ds(i, _L)]
                xf = (x - x0) * inv_dx
                bi = xf.astype(jnp.int32)
                t = xf - bi.astype(jnp.float32)
                c0 = plsc.load_gather(c0tab, [bi])
                c1 = plsc.load_gather(c1tab, [bi])
                c2 = plsc.load_gather(c2tab, [bi])
                c3 = c3v.at[bi].get(mode="promise_in_bounds")
                t2 = t * t
                dst[pl.ds(i, _L)] = (c0 + c2 * t2) + (c1 + c3 * t2) * t

        xins = (xin0, xin1)
        youts = (yout0, yout1)
        sem_ins = (sem_in0, sem_in1)
        sem_outs = (sem_out0, sem_out1)

        def in_slice(g):
            return xs_hbm.at[pl.ds(base + g * _CHUNK, _CHUNK)]

        def out_slice(g):
            return out_hbm.at[pl.ds(base + g * _CHUNK, _CHUNK)]

        pltpu.async_copy(in_slice(0), xin0, sem_in0)
        pltpu.async_copy(in_slice(1), xin1, sem_in1)
        for g in range(n_chunks):
            bb = g % 2
            pltpu.make_async_copy(in_slice(g), xins[bb], sem_ins[bb]).wait()
            if g >= 2:
                pltpu.make_async_copy(
                    youts[bb], out_slice(g - 2), sem_outs[bb]).wait()
            compute(xins[bb], youts[bb])
            pltpu.async_copy(youts[bb], out_slice(g), sem_outs[bb])
            if g + 2 < n_chunks:
                pltpu.async_copy(in_slice(g + 2), xins[bb], sem_ins[bb])
        for g in (n_chunks - 2, n_chunks - 1):
            bb = g % 2
            pltpu.make_async_copy(
                youts[bb], out_slice(g), sem_outs[bb]).wait()

    return body


@functools.partial(jax.jit, static_argnums=(3,))
def _interp(xs, x16, y16, npts):
    mesh = plsc.VectorSubcoreMesh(core_axis_name="c", subcore_axis_name="s")
    run = pl.kernel(
        _make_body(npts),
        out_type=jax.ShapeDtypeStruct(xs.shape, xs.dtype),
        mesh=mesh,
        compiler_params=pltpu.CompilerParams(needs_layout_passes=False),
        scratch_types=[
            pltpu.VMEM((9,), jnp.float32),
            pltpu.VMEM((9,), jnp.float32),
            pltpu.VMEM((_L,), jnp.float32),
            pltpu.VMEM((_L,), jnp.float32),
            pltpu.VMEM((_L,), jnp.float32),
            pltpu.VMEM((_L,), jnp.float32),
            pltpu.VMEM((_CHUNK,), jnp.float32),
            pltpu.VMEM((_CHUNK,), jnp.float32),
            pltpu.VMEM((_CHUNK,), jnp.float32),
            pltpu.VMEM((_CHUNK,), jnp.float32),
            pltpu.SemaphoreType.DMA,
            pltpu.SemaphoreType.DMA,
            pltpu.SemaphoreType.DMA,
            pltpu.SemaphoreType.DMA,
        ],
    )
    return run(xs, x16, y16)


def kernel(xs, x_ctl, y_ctl, A):
    del A
    return _interp(xs, x_ctl, y_ctl, x_ctl.shape[0])


# speedup vs baseline: 1.0540x; 1.0540x over previous
import functools

import jax
import jax.numpy as jnp
from jax import lax
from jax.experimental import pallas as pl
from jax.experimental.pallas import tpu as pltpu
from jax.experimental.pallas import tpu_sc as plsc

_NC = 2
_NS = 16
_NW = _NC * _NS
_L = 16
_CHUNK = 16384
_UNROLL = 8


def _splat(tab_ref, idx):
    return plsc.load_gather(tab_ref, [jnp.full((_L,), idx, jnp.int32)])


def _make_body(npts):
    def body(xs_hbm, xctl_hbm, yctl_hbm, out_hbm,
             xtab, ytab, c0tab, c1tab, c2tab, c3tab,
             xin0, xin1, yout0, yout1,
             sem_in0, sem_in1, sem_out0, sem_out1):
        n = out_hbm.shape[0]
        per_worker = n // _NW
        n_chunks = per_worker // _CHUNK
        wid = lax.axis_index("s") * _NC + lax.axis_index("c")
        base = wid * per_worker

        pltpu.sync_copy(xctl_hbm, xtab)
        pltpu.sync_copy(yctl_hbm, ytab)

        k = lax.iota(jnp.int32, _L)
        kc = jnp.minimum(k, npts - 1)
        kp1 = jnp.minimum(k + 1, npts - 1)
        xv = plsc.load_gather(xtab, [kc])
        yv = plsc.load_gather(ytab, [kc])
        xn = plsc.load_gather(xtab, [kp1])
        yn = plsc.load_gather(ytab, [kp1])
        s = (yn - yv) / (xn - xv)
        c0tab[...] = s
        s_last = _splat(c0tab, npts - 2)
        s = jnp.where(k >= npts - 1, s_last, s)
        c0tab[...] = s
        s_prev = plsc.load_gather(c0tab, [jnp.maximum(k - 1, 0)])
        m = (s + s_prev) * 0.5
        c0tab[...] = m
        m_next = plsc.load_gather(c0tab, [kp1])

        x0 = _splat(xtab, 0)
        dx = _splat(xtab, 1) - x0
        inv_dx = 1.0 / dx

        a = m * dx
        b = m_next * dx
        dy = yn - yv
        c0v = yv
        c1v = a
        c0tab[...] = c0v
        c1tab[...] = c1v
        pad_ok = k < npts - 1
        c2tab[...] = jnp.where(pad_ok, 3.0 * dy - 2.0 * a - b, 0.0)
        c3tab[...] = jnp.where(pad_ok, a + b - 2.0 * dy, 0.0)
        c2v = c2tab[...]
        c3v = c3tab[...]
        bmax = jnp.full((_L,), float(npts - 2), jnp.float32)

        def compute(src, dst):
            @plsc.parallel_loop(0, _CHUNK, step=_L, unroll=_UNROLL)
            def _vecs(i):
                x = src[pl.ds(i, _L)]
                xf = (x - x0) * inv_dx
                bi = xf.astype(jnp.int32)
                t = xf - bi.astype(jnp.float32)
                c0 = plsc.load_gather(c0tab, [bi])
                c1 = plsc.load_gather(c1tab, [bi])
                c2 = plsc.load_gather(c2tab, [bi])
                c3 = c3v.at[bi].get(mode="promise_in_bounds")
                dst[pl.ds(i, _L)] = ((c3 * t + c2) * t + c1) * t + c0

        xins = (xin0, xin1)
        youts = (yout0, yout1)
        sem_ins = (sem_in0, sem_in1)
        sem_outs = (sem_out0, sem_out1)

        def in_slice(g):
            return xs_hbm.at[pl.ds(base + g * _CHUNK, _CHUNK)]

        def out_slice(g):
            return out_hbm.at[pl.ds(base + g * _CHUNK, _CHUNK)]

        pltpu.async_copy(in_slice(0), xin0, sem_in0)
        pltpu.async_copy(in_slice(1), xin1, sem_in1)
        for g in range(n_chunks):
            bb = g % 2
            pltpu.make_async_copy(in_slice(g), xins[bb], sem_ins[bb]).wait()
            if g >= 2:
                pltpu.make_async_copy(
                    youts[bb], out_slice(g - 2), sem_outs[bb]).wait()
            compute(xins[bb], youts[bb])
            pltpu.async_copy(youts[bb], out_slice(g), sem_outs[bb])
            if g + 2 < n_chunks:
                pltpu.async_copy(in_slice(g + 2), xins[bb], sem_ins[bb])
        for g in (n_chunks - 2, n_chunks - 1):
            bb = g % 2
            pltpu.make_async_copy(
                youts[bb], out_slice(g), sem_outs[bb]).wait()

    return body


@functools.partial(jax.jit, static_argnums=(3,))
def _interp(xs, x16, y16, npts):
    mesh = plsc.VectorSubcoreMesh(core_axis_name="c", subcore_axis_name="s")
    run = pl.kernel(
        _make_body(npts),
        out_type=jax.ShapeDtypeStruct(xs.shape, xs.dtype),
        mesh=mesh,
        compiler_params=pltpu.CompilerParams(needs_layout_passes=False),
        scratch_types=[
            pltpu.VMEM((9,), jnp.float32),
            pltpu.VMEM((9,), jnp.float32),
            pltpu.VMEM((_L,), jnp.float32),
            pltpu.VMEM((_L,), jnp.float32),
            pltpu.VMEM((_L,), jnp.float32),
            pltpu.VMEM((_L,), jnp.float32),
            pltpu.VMEM((_CHUNK,), jnp.float32),
            pltpu.VMEM((_CHUNK,), jnp.float32),
            pltpu.VMEM((_CHUNK,), jnp.float32),
            pltpu.VMEM((_CHUNK,), jnp.float32),
            pltpu.SemaphoreType.DMA,
            pltpu.SemaphoreType.DMA,
            pltpu.SemaphoreType.DMA,
            pltpu.SemaphoreType.DMA,
        ],
    )
    return run(xs, x16, y16)


def kernel(xs, x_ctl, y_ctl, A):
    del A
    return _interp(xs, x_ctl, y_ctl, x_ctl.shape[0])
